# baseline (speedup 1.0000x reference)
import dataclasses
import functools

import jax
import jax.numpy as jnp
from jax import lax
from jax.experimental import pallas as pl
from jax.experimental.pallas import tpu as pltpu
from jax.experimental.pallas import tpu_sc as plsc

N = 10000
D = 256
ED = 16
NC = 2
NS = 16
L = 16

_MESH = dict(mesh=plsc.VectorSubcoreMesh(core_axis_name="c", subcore_axis_name="s"))

_CP = pltpu.CompilerParams()
if "needs_layout_passes" in pltpu.CompilerParams.__dataclass_fields__:
    _CP = dataclasses.replace(_CP, needs_layout_passes=False)


def _make_idx_kernel(EP):
    per_w = EP // (NC * NS)
    n_ch = per_w // 128

    @functools.partial(
        pl.kernel,
        out_type=(
            jax.ShapeDtypeStruct((EP,), jnp.int32),
            jax.ShapeDtypeStruct((EP,), jnp.int32),
        ),
        scratch_types=[
            pltpu.VMEM((128,), jnp.int32),
            pltpu.VMEM((128,), jnp.int32),
            pltpu.VMEM((128,), jnp.int32),
        ],
        **_MESH,
    )
    def idx_kernel(src_h, dst_h, nt_h, tdst_h, gsrc_h, dstb, srcb, tdb):
        wid = lax.axis_index("s") * NC + lax.axis_index("c")
        w0 = wid * per_w

        @pl.loop(0, n_ch)
        def _(ch):
            b = w0 + ch * 128
            pltpu.sync_copy(dst_h.at[pl.ds(b, 128)], dstb)
            pltpu.sync_copy(src_h.at[pl.ds(b, 128)], srcb)
            pltpu.sync_copy(nt_h.at[dstb], tdb)
            pltpu.sync_copy(tdb, tdst_h.at[pl.ds(b, 128)])
            for v in range(8):
                sl = pl.ds(v * L, L)
                tdb[sl] = tdb[sl] * N + srcb[sl]
            pltpu.sync_copy(tdb, gsrc_h.at[pl.ds(b, 128)])

    return idx_kernel


def _bnodes_body(h_ref, nt_ref, wl_ref, bl_ref, wr_ref, br_ref, ws_ref, bs_ref,
                 xl_ref, xr_ref, res_ref):
    hb = h_ref[...]
    ntb = nt_ref[...]
    xr = []
    rs = []
    for b in range(3):
        xl_ref[b] = jnp.dot(hb, wl_ref[b], preferred_element_type=jnp.float32) + bl_ref[b][None, :]
        xr.append(jnp.dot(hb, wr_ref[b], preferred_element_type=jnp.float32) + br_ref[b][None, :])
        rs.append(jnp.dot(hb, ws_ref[b], preferred_element_type=jnp.float32) + bs_ref[b][None, :])
    xr_ref[...] = jnp.where(ntb == 0, xr[0], jnp.where(ntb == 1, xr[1], xr[2]))
    res_ref[...] = jnp.where(ntb == 0, rs[0], jnp.where(ntb == 1, rs[1], rs[2]))


def _run_bnodes(h, nt2, Wl3, bl3, Wr3, br3, Wres3, bias3):
    BN = 1000
    full = lambda shp: pl.BlockSpec(shp, lambda i: tuple(0 for _ in shp))
    return pl.pallas_call(
        _bnodes_body,
        grid=(N // BN,),
        in_specs=[
            pl.BlockSpec((BN, D), lambda i: (i, 0)),
            pl.BlockSpec((BN, 1), lambda i: (i, 0)),
            full((3, D, D)), full((3, D)), full((3, D, D)), full((3, D)),
            full((3, D, D)), full((3, D)),
        ],
        out_specs=[
            pl.BlockSpec((3, BN, D), lambda i: (0, i, 0)),
            pl.BlockSpec((BN, D), lambda i: (i, 0)),
            pl.BlockSpec((BN, D), lambda i: (i, 0)),
        ],
        out_shape=[
            jax.ShapeDtypeStruct((3, N, D), jnp.float32),
            jax.ShapeDtypeStruct((N, D), jnp.float32),
            jax.ShapeDtypeStruct((N, D), jnp.float32),
        ],
    )(h, nt2, Wl3, bl3, Wr3, br3, Wres3, bias3)


def _bedges_body(ea_ref, td_ref, we_ref, e_out):
    eab = ea_ref[...]
    tdb = td_ref[...]
    es = [jnp.dot(eab, we_ref[b], preferred_element_type=jnp.float32) for b in range(3)]
    e_out[...] = jnp.where(tdb == 0, es[0], jnp.where(tdb == 1, es[1], es[2]))


def _run_bedges(ea_pad, td2, We3, EP):
    BE = 2048
    return pl.pallas_call(
        _bedges_body,
        grid=(EP // BE,),
        in_specs=[
            pl.BlockSpec((BE, ED), lambda i: (i, 0)),
            pl.BlockSpec((BE, 1), lambda i: (i, 0)),
            pl.BlockSpec((3, ED, D), lambda i: (0, 0, 0)),
        ],
        out_specs=pl.BlockSpec((BE, D), lambda i: (i, 0)),
        out_shape=jax.ShapeDtypeStruct((EP, D), jnp.float32),
    )(ea_pad, td2, We3)


def _make_gather_kernel(EP):
    per_w = EP // (NC * NS)
    CH = 80
    n_ch = per_w // CH

    @functools.partial(
        pl.kernel,
        out_type=(
            jax.ShapeDtypeStruct((EP, D), jnp.float32),
            jax.ShapeDtypeStruct((EP, D), jnp.float32),
        ),
        scratch_types=[
            pltpu.VMEM((per_w,), jnp.int32),
            pltpu.VMEM((per_w,), jnp.int32),
            pltpu.VMEM((CH, D), jnp.float32),
            pltpu.VMEM((CH, D), jnp.float32),
            pltpu.VMEM((CH, D), jnp.float32),
            pltpu.VMEM((CH, D), jnp.float32),
            pltpu.SemaphoreType.DMA,
            pltpu.SemaphoreType.DMA,
            pltpu.SemaphoreType.DMA,
            pltpu.SemaphoreType.DMA,
        ],
        compiler_params=_CP,
        **_MESH,
    )
    def gather_kernel(xl_h, xr_h, gsrc_h, dst_h, gxl_h, gxr_h,
                      gss, dss, a0, b0, a1, b1, sg0, sg1, sw0, sw1):
        wid = lax.axis_index("s") * NC + lax.axis_index("c")
        w0 = wid * per_w
        pltpu.sync_copy(gsrc_h.at[pl.ds(w0, per_w)], gss)
        pltpu.sync_copy(dst_h.at[pl.ds(w0, per_w)], dss)

        bufs = ((a0, b0), (a1, b1))
        sgs = (sg0, sg1)
        sws = (sw0, sw1)
        outs = (gxl_h, gxr_h)

        @pl.loop(0, n_ch, step=2)
        def _(ch0):
            for b in range(2):
                ch = ch0 + b
                off = ch * CH
                gout = w0 + off
                bb = bufs[b]

                @pl.when(ch0 >= 2)
                def _():
                    for t in range(2):
                        pltpu.make_async_copy(
                            outs[t].at[pl.ds(0, CH)], bb[t], sws[b]).wait()

                h0 = pltpu.async_copy(xl_h.at[gss.at[pl.ds(off, CH)]], bb[0], sgs[b])
                h1 = pltpu.async_copy(xr_h.at[dss.at[pl.ds(off, CH)]], bb[1], sgs[b])
                h0.wait()
                h1.wait()
                for t in range(2):
                    pltpu.async_copy(bb[t], outs[t].at[pl.ds(gout, CH)], sws[b])

        for b in range(2):
            for t in range(2):
                pltpu.make_async_copy(
                    outs[t].at[pl.ds(0, CH)], bufs[b][t], sws[b]).wait()

    return gather_kernel


def _make_c2(EP, E):
    BE = 2048

    def c2_body(xg_ref, xrg_ref, e_ref, td_ref, att_ref,
                pa_ref, pb_ref, ex_ref):
        i = pl.program_id(0)
        td = td_ref[...]
        att = jnp.where(
            td == 0, att_ref[0][None, :],
            jnp.where(td == 1, att_ref[1][None, :], att_ref[2][None, :]))
        xg = xg_ref[...]
        m = xg + xrg_ref[...] + e_ref[...]
        m = jnp.maximum(m, m * 0.2)
        alpha = jnp.sum(m * att, axis=1, keepdims=True)
        ids = i * BE + lax.broadcasted_iota(jnp.int32, (BE, 1), 0)
        ex = jnp.where(ids < E, jnp.exp(alpha), 0.0)
        ex_ref[...] = ex
        pa_ref[...] = xg[:, :128] * ex
        pb_ref[...] = xg[:, 128:] * ex

    def run(xg, xrg, e_o, td2, att3):
        fullw = pl.BlockSpec((BE, D), lambda i: (i, 0))
        half = pl.BlockSpec((BE, 128), lambda i: (i, 0))
        return pl.pallas_call(
            c2_body,
            grid=(EP // BE,),
            in_specs=[fullw, fullw, fullw,
                      pl.BlockSpec((BE, 1), lambda i: (i, 0)),
                      pl.BlockSpec((3, D), lambda i: (0, 0))],
            out_specs=[half, half, pl.BlockSpec((BE, 1), lambda i: (i, 0))],
            out_shape=[
                jax.ShapeDtypeStruct((EP, 128), jnp.float32),
                jax.ShapeDtypeStruct((EP, 128), jnp.float32),
                jax.ShapeDtypeStruct((EP, 1), jnp.float32),
            ],
        )(xg, xrg, e_o, td2, att3)

    return run


def _make_accum_kernel(EP):
    per_s = EP // NS
    CH = 64
    n_ch = per_s // CH

    @functools.partial(
        pl.kernel,
        out_type=(
            jax.ShapeDtypeStruct((NC, N, 128), jnp.float32),
            jax.ShapeDtypeStruct((N,), jnp.float32),
        ),
        scratch_types=[
            pltpu.VMEM((n_ch, CH), jnp.int32),
            pltpu.VMEM((CH, 128), jnp.float32),
            pltpu.VMEM((CH, 128), jnp.float32),
            pltpu.VMEM((CH,), jnp.float32),
            pltpu.VMEM((CH,), jnp.float32),
            pltpu.VMEM((64, 128), jnp.float32),
            pltpu.VMEM((640,), jnp.float32),
            pltpu.SemaphoreType.DMA,
            pltpu.SemaphoreType.DMA,
            pltpu.SemaphoreType.DMA,
            pltpu.SemaphoreType.DMA,
            pltpu.VMEM_SHARED((N, 128), jnp.float32),
            pltpu.VMEM_SHARED((N,), jnp.float32),
        ],
        compiler_params=_CP,
        **_MESH,
    )
    def accum_kernel(pa_h, pb_h, dst3_h, ex_h, out_h, den_h,
                     dss, r0buf, r1buf, e0buf, e1buf, zb, zbd,
                     sr0, sr1, ss0, ss1,
                     acc_sh, den_sh):
        cid = lax.axis_index("c")
        sid = lax.axis_index("s")
        s0 = sid * per_s
        pltpu.sync_copy(dst3_h.at[sid], dss)

        @pl.when(sid == 0)
        def _():
            @pl.loop(0, 64)
            def _(r):
                for k in range(8):
                    zb[r, pl.ds(k * L, L)] = jnp.zeros((L,), jnp.float32)

            @pl.loop(0, 640 // L)
            def _(i):
                zbd[pl.ds(i * L, L)] = jnp.zeros((L,), jnp.float32)

            @pl.loop(0, 156)
            def _(i):
                pltpu.sync_copy(zb, acc_sh.at[pl.ds(i * 64, 64)])

            pltpu.sync_copy(zb.at[pl.ds(0, 16)], acc_sh.at[pl.ds(9984, 16)])

            @pl.loop(0, 15)
            def _(i):
                pltpu.sync_copy(zbd, den_sh.at[pl.ds(i * 640, 640)])

            pltpu.sync_copy(zbd.at[pl.ds(0, 400)], den_sh.at[pl.ds(9600, 400)])

        plsc.subcore_barrier()

        rbufs = (r0buf, r1buf)
        ebufs = (e0buf, e1buf)
        srs = (sr0, sr1)
        sss = (ss0, ss1)

        @pl.loop(0, n_ch, step=2)
        def _(ch0):
            for b in range(2):
                ch = ch0 + b
                gbase = s0 + ch * CH
                rb = rbufs[b]
                eb = ebufs[b]

                @pl.when(ch0 >= 2)
                def _():
                    pltpu.make_async_copy(pa_h.at[pl.ds(0, CH)], rb, sss[b]).wait()
                    pltpu.make_async_copy(ex_h.at[pl.ds(0, CH)], eb, sss[b]).wait()

                @pl.when(cid == 0)
                def _():
                    pltpu.async_copy(pa_h.at[pl.ds(gbase, CH)], rb, srs[b])

                @pl.when(cid == 1)
                def _():
                    pltpu.async_copy(pb_h.at[pl.ds(gbase, CH)], rb, srs[b])

                pltpu.async_copy(ex_h.at[pl.ds(gbase, CH)], eb, srs[b])
                pltpu.make_async_copy(pa_h.at[pl.ds(0, CH)], rb, srs[b]).wait()
                pltpu.make_async_copy(ex_h.at[pl.ds(0, CH)], eb, srs[b]).wait()

                idx = dss.at[ch]
                pltpu.async_copy(rb, acc_sh.at[idx], sss[b], add=True)
                pltpu.async_copy(eb, den_sh.at[idx], sss[b], add=True)

        for b in range(2):
            pltpu.make_async_copy(pa_h.at[pl.ds(0, CH)], rbufs[b], sss[b]).wait()
            pltpu.make_async_copy(ex_h.at[pl.ds(0, CH)], ebufs[b], sss[b]).wait()

        plsc.subcore_barrier()

        @pl.when(sid < 15)
        def _():
            pltpu.sync_copy(acc_sh.at[pl.ds(sid * 640, 640)],
                            out_h.at[cid, pl.ds(sid * 640, 640)])

        @pl.when(sid == 15)
        def _():
            pltpu.sync_copy(acc_sh.at[pl.ds(9600, 400)],
                            out_h.at[cid, pl.ds(9600, 400)])

        @pl.when((sid == 0) & (cid == 0))
        def _():
            pltpu.sync_copy(den_sh, den_h)

    return accum_kernel


def _final_body(acc_ref, den_ref, res_ref, o_ref):
    den = den_ref[...] + 1e-16
    o_ref[:, :128] = jax.nn.relu(acc_ref[0] / den + res_ref[:, :128])
    o_ref[:, 128:] = jax.nn.relu(acc_ref[1] / den + res_ref[:, 128:])


def _run_final(acc2, den2, res):
    BN = 1000
    return pl.pallas_call(
        _final_body,
        grid=(N // BN,),
        in_specs=[
            pl.BlockSpec((NC, BN, 128), lambda i: (0, i, 0)),
            pl.BlockSpec((BN, 1), lambda i: (i, 0)),
            pl.BlockSpec((BN, D), lambda i: (i, 0)),
        ],
        out_specs=pl.BlockSpec((BN, D), lambda i: (i, 0)),
        out_shape=jax.ShapeDtypeStruct((N, D), jnp.float32),
    )(acc2, den2, res)


def kernel(h, edge_index, edge_attr, node_type, params):
    src = edge_index[0].astype(jnp.int32)
    dst = edge_index[1].astype(jnp.int32)
    nt = node_type.astype(jnp.int32)
    E = src.shape[0]
    EP = ((E + 4095) // 4096) * 4096
    pad = EP - E

    src_p = jnp.concatenate([src, jnp.zeros((pad,), jnp.int32)])
    dst_p = jnp.concatenate([dst, jnp.zeros((pad,), jnp.int32)])
    ea_p = jnp.concatenate([edge_attr, jnp.zeros((pad, ED), jnp.float32)])

    names = ("var", "red", "irr")
    Wl3 = jnp.stack([params[k]["Wl"] for k in names])
    bl3 = jnp.stack([params[k]["bl"] for k in names])
    Wr3 = jnp.stack([params[k]["Wr"] for k in names])
    br3 = jnp.stack([params[k]["br"] for k in names])
    We3 = jnp.stack([params[k]["We"] for k in names])
    att3 = jnp.stack([params[k]["att"] for k in names])
    Wres3 = jnp.stack([params[k]["Wres"] for k in names])
    bias3 = jnp.stack([params[k]["bias"] for k in names])

    tdst, gsrc = _make_idx_kernel(EP)(src_p, dst_p, nt)
    xl3, xr, res = _run_bnodes(
        h, nt.reshape(N, 1), Wl3, bl3, Wr3, br3, Wres3, bias3)
    xl = xl3.reshape(3 * N, D)
    e_o = _run_bedges(ea_p, tdst.reshape(EP, 1), We3, EP)

    xg, xrg = _make_gather_kernel(EP)(xl, xr, gsrc, dst_p)
    pa, pb, ex2 = _make_c2(EP, E)(xg, xrg, e_o, tdst.reshape(EP, 1), att3)

    dst3 = dst_p.reshape(NS, EP // NS // 64, 64)
    acc2, den = _make_accum_kernel(EP)(pa, pb, dst3, ex2.reshape(EP))
    return _run_final(acc2, den.reshape(N, 1), res)


# speedup vs baseline: 1.1731x; 1.1731x over previous
import dataclasses
import functools

import jax
import jax.numpy as jnp
from jax import lax
from jax.experimental import pallas as pl
from jax.experimental.pallas import tpu as pltpu
from jax.experimental.pallas import tpu_sc as plsc

N = 10000
D = 256
ED = 16
NC = 2
NS = 16
L = 16

_MESH = dict(mesh=plsc.VectorSubcoreMesh(core_axis_name="c", subcore_axis_name="s"))

_CP = pltpu.CompilerParams()
if "needs_layout_passes" in pltpu.CompilerParams.__dataclass_fields__:
    _CP = dataclasses.replace(_CP, needs_layout_passes=False)


def _make_idx_kernel(EP):
    per_w = EP // (NC * NS)
    n_ch = per_w // 128

    @functools.partial(
        pl.kernel,
        out_type=(
            jax.ShapeDtypeStruct((EP,), jnp.int32),
            jax.ShapeDtypeStruct((EP,), jnp.int32),
        ),
        scratch_types=[
            pltpu.VMEM((per_w,), jnp.int32),
            pltpu.VMEM((per_w,), jnp.int32),
            pltpu.VMEM((per_w,), jnp.int32),
            pltpu.VMEM((per_w,), jnp.int32),
            pltpu.VMEM((128,), jnp.int32),
            pltpu.VMEM((128,), jnp.int32),
            pltpu.SemaphoreType.DMA,
            pltpu.SemaphoreType.DMA,
        ],
        compiler_params=_CP,
        **_MESH,
    )
    def idx_kernel(src_h, dst_h, nt_h, tdst_h, gsrc_h,
                   dss, sss, tds, gss, td0, td1, sg0, sg1):
        wid = lax.axis_index("s") * NC + lax.axis_index("c")
        w0 = wid * per_w
        pltpu.sync_copy(dst_h.at[pl.ds(w0, per_w)], dss)
        pltpu.sync_copy(src_h.at[pl.ds(w0, per_w)], sss)
        tdb = (td0, td1)
        sgs = (sg0, sg1)

        @pl.loop(0, n_ch, step=2)
        def _(ch0):
            for b in range(2):
                off = (ch0 + b) * 128
                pltpu.async_copy(
                    nt_h.at[dss.at[pl.ds(off, 128)]], tdb[b], sgs[b])
            for b in range(2):
                off = (ch0 + b) * 128
                pltpu.make_async_copy(
                    nt_h.at[pl.ds(0, 128)], tdb[b], sgs[b]).wait()
                for v in range(8):
                    sl = pl.ds(off + v * L, L)
                    t = tdb[b][pl.ds(v * L, L)]
                    tds[sl] = t
                    gss[sl] = t * N + sss[sl]

        pltpu.sync_copy(tds, tdst_h.at[pl.ds(w0, per_w)])
        pltpu.sync_copy(gss, gsrc_h.at[pl.ds(w0, per_w)])

    return idx_kernel


def _bnodes_body(h_ref, nt_ref, wl_ref, bl_ref, wr_ref, br_ref, ws_ref, bs_ref,
                 xl_ref, xr_ref, res_ref):
    hb = h_ref[...]
    ntb = nt_ref[...]
    xr = []
    rs = []
    for b in range(3):
        xl_ref[b] = jnp.dot(hb, wl_ref[b], preferred_element_type=jnp.float32) + bl_ref[b][None, :]
        xr.append(jnp.dot(hb, wr_ref[b], preferred_element_type=jnp.float32) + br_ref[b][None, :])
        rs.append(jnp.dot(hb, ws_ref[b], preferred_element_type=jnp.float32) + bs_ref[b][None, :])
    xr_ref[...] = jnp.where(ntb == 0, xr[0], jnp.where(ntb == 1, xr[1], xr[2]))
    res_ref[...] = jnp.where(ntb == 0, rs[0], jnp.where(ntb == 1, rs[1], rs[2]))


def _run_bnodes(h, nt2, Wl3, bl3, Wr3, br3, Wres3, bias3):
    BN = 1000
    full = lambda shp: pl.BlockSpec(shp, lambda i: tuple(0 for _ in shp))
    return pl.pallas_call(
        _bnodes_body,
        grid=(N // BN,),
        in_specs=[
            pl.BlockSpec((BN, D), lambda i: (i, 0)),
            pl.BlockSpec((BN, 1), lambda i: (i, 0)),
            full((3, D, D)), full((3, D)), full((3, D, D)), full((3, D)),
            full((3, D, D)), full((3, D)),
        ],
        out_specs=[
            pl.BlockSpec((3, BN, D), lambda i: (0, i, 0)),
            pl.BlockSpec((BN, D), lambda i: (i, 0)),
            pl.BlockSpec((BN, D), lambda i: (i, 0)),
        ],
        out_shape=[
            jax.ShapeDtypeStruct((3, N, D), jnp.float32),
            jax.ShapeDtypeStruct((N, D), jnp.float32),
            jax.ShapeDtypeStruct((N, D), jnp.float32),
        ],
    )(h, nt2, Wl3, bl3, Wr3, br3, Wres3, bias3)


def _make_gather_kernel(EP):
    per_w = EP // (NC * NS)
    CH = 40
    n_ch = per_w // CH
    NB = 4

    @functools.partial(
        pl.kernel,
        out_type=(
            jax.ShapeDtypeStruct((EP, D), jnp.float32),
            jax.ShapeDtypeStruct((EP, D), jnp.float32),
        ),
        scratch_types=(
            [pltpu.VMEM((per_w,), jnp.int32)] * 2
            + [pltpu.VMEM((CH, D), jnp.float32)] * (2 * NB)
            + [pltpu.SemaphoreType.DMA] * (2 * NB)
        ),
        compiler_params=_CP,
        **_MESH,
    )
    def gather_kernel(xl_h, xr_h, gsrc_h, dst_h, gxl_h, gxr_h,
                      gss, dss, *bufsem):
        xlb = bufsem[0:NB]
        xrb = bufsem[NB:2 * NB]
        sg = bufsem[2 * NB:3 * NB]
        sw = bufsem[3 * NB:4 * NB]
        wid = lax.axis_index("s") * NC + lax.axis_index("c")
        w0 = wid * per_w
        pltpu.sync_copy(gsrc_h.at[pl.ds(w0, per_w)], gss)
        pltpu.sync_copy(dst_h.at[pl.ds(w0, per_w)], dss)

        def issue_g(k, s):
            off = k * CH
            pltpu.async_copy(xl_h.at[gss.at[pl.ds(off, CH)]], xlb[s], sg[s])
            pltpu.async_copy(xr_h.at[dss.at[pl.ds(off, CH)]], xrb[s], sg[s])

        issue_g(0, 0)
        issue_g(1, 1)

        @pl.loop(0, n_ch, step=NB)
        def _(k0):
            for j in range(NB):
                k = k0 + j
                s = j
                pltpu.make_async_copy(gxl_h.at[pl.ds(0, CH)], xlb[s], sg[s]).wait()
                pltpu.make_async_copy(gxl_h.at[pl.ds(0, CH)], xrb[s], sg[s]).wait()
                gout = w0 + k * CH
                pltpu.async_copy(xlb[s], gxl_h.at[pl.ds(gout, CH)], sw[s])
                pltpu.async_copy(xrb[s], gxr_h.at[pl.ds(gout, CH)], sw[s])
                s1 = (j + 3) % NB

                @pl.when(k >= 1)
                def _():
                    pltpu.make_async_copy(gxl_h.at[pl.ds(0, CH)], xlb[s1], sw[s1]).wait()
                    pltpu.make_async_copy(gxl_h.at[pl.ds(0, CH)], xrb[s1], sw[s1]).wait()

                s2 = (j + 2) % NB

                @pl.when(k + 2 < n_ch)
                def _():
                    issue_g(k + 2, s2)

        s_last = (n_ch - 1) % NB
        pltpu.make_async_copy(gxl_h.at[pl.ds(0, CH)], xlb[s_last], sw[s_last]).wait()
        pltpu.make_async_copy(gxl_h.at[pl.ds(0, CH)], xrb[s_last], sw[s_last]).wait()

    return gather_kernel


def _make_c2(EP, E):
    BE = 2048

    def c2_body(xg_ref, xrg_ref, ea_ref, td_ref, att_ref, we_ref,
                pa_ref, pb_ref, ex_ref):
        i = pl.program_id(0)
        td = td_ref[...]
        att = jnp.where(
            td == 0, att_ref[0][None, :],
            jnp.where(td == 1, att_ref[1][None, :], att_ref[2][None, :]))
        eab = ea_ref[...]
        es = [jnp.dot(eab, we_ref[b], preferred_element_type=jnp.float32)
              for b in range(3)]
        e = jnp.where(td == 0, es[0], jnp.where(td == 1, es[1], es[2]))
        xg = xg_ref[...]
        m = xg + xrg_ref[...] + e
        m = jnp.maximum(m, m * 0.2)
        alpha = jnp.sum(m * att, axis=1, keepdims=True)
        ids = i * BE + lax.broadcasted_iota(jnp.int32, (BE, 1), 0)
        ex = jnp.where(ids < E, jnp.exp(alpha), 0.0)
        ex_ref[...] = ex
        pa_ref[...] = xg[:, :128] * ex
        pb_ref[...] = xg[:, 128:] * ex

    def run(xg, xrg, ea_p, td2, att3, We3):
        fullw = pl.BlockSpec((BE, D), lambda i: (i, 0))
        half = pl.BlockSpec((BE, 128), lambda i: (i, 0))
        return pl.pallas_call(
            c2_body,
            grid=(EP // BE,),
            in_specs=[fullw, fullw,
                      pl.BlockSpec((BE, ED), lambda i: (i, 0)),
                      pl.BlockSpec((BE, 1), lambda i: (i, 0)),
                      pl.BlockSpec((3, D), lambda i: (0, 0)),
                      pl.BlockSpec((3, ED, D), lambda i: (0, 0, 0))],
            out_specs=[half, half, pl.BlockSpec((BE, 1), lambda i: (i, 0))],
            out_shape=[
                jax.ShapeDtypeStruct((EP, 128), jnp.float32),
                jax.ShapeDtypeStruct((EP, 128), jnp.float32),
                jax.ShapeDtypeStruct((EP, 1), jnp.float32),
            ],
        )(xg, xrg, ea_p, td2, att3, We3)

    return run


def _make_accum_kernel(EP):
    per_s = EP // NS
    CH = 64
    n_ch = per_s // CH

    @functools.partial(
        pl.kernel,
        out_type=(
            jax.ShapeDtypeStruct((NC, N, 128), jnp.float32),
            jax.ShapeDtypeStruct((N,), jnp.float32),
        ),
        scratch_types=[
            pltpu.VMEM((n_ch, CH), jnp.int32),
            pltpu.VMEM((CH, 128), jnp.float32),
            pltpu.VMEM((CH, 128), jnp.float32),
            pltpu.VMEM((CH,), jnp.float32),
            pltpu.VMEM((CH,), jnp.float32),
            pltpu.VMEM((64, 128), jnp.float32),
            pltpu.VMEM((640,), jnp.float32),
            pltpu.SemaphoreType.DMA,
            pltpu.SemaphoreType.DMA,
            pltpu.SemaphoreType.DMA,
            pltpu.SemaphoreType.DMA,
            pltpu.VMEM_SHARED((N, 128), jnp.float32),
            pltpu.VMEM_SHARED((N,), jnp.float32),
        ],
        compiler_params=_CP,
        **_MESH,
    )
    def accum_kernel(pa_h, pb_h, dst3_h, ex_h, out_h, den_h,
                     dss, r0buf, r1buf, e0buf, e1buf, zb, zbd,
                     sr0, sr1, ss0, ss1,
                     acc_sh, den_sh):
        cid = lax.axis_index("c")
        sid = lax.axis_index("s")
        s0 = sid * per_s
        pltpu.sync_copy(dst3_h.at[sid], dss)

        @pl.when(sid == 0)
        def _():
            @pl.loop(0, 64)
            def _(r):
                for k in range(8):
                    zb[r, pl.ds(k * L, L)] = jnp.zeros((L,), jnp.float32)

            @pl.loop(0, 640 // L)
            def _(i):
                zbd[pl.ds(i * L, L)] = jnp.zeros((L,), jnp.float32)

            @pl.loop(0, 156)
            def _(i):
                pltpu.sync_copy(zb, acc_sh.at[pl.ds(i * 64, 64)])

            pltpu.sync_copy(zb.at[pl.ds(0, 16)], acc_sh.at[pl.ds(9984, 16)])

            @pl.loop(0, 15)
            def _(i):
                pltpu.sync_copy(zbd, den_sh.at[pl.ds(i * 640, 640)])

            pltpu.sync_copy(zbd.at[pl.ds(0, 400)], den_sh.at[pl.ds(9600, 400)])

        plsc.subcore_barrier()

        rbufs = (r0buf, r1buf)
        ebufs = (e0buf, e1buf)
        srs = (sr0, sr1)
        sss = (ss0, ss1)

        @pl.loop(0, n_ch, step=2)
        def _(ch0):
            for b in range(2):
                ch = ch0 + b
                gbase = s0 + ch * CH
                rb = rbufs[b]
                eb = ebufs[b]

                @pl.when(ch0 >= 2)
                def _():
                    pltpu.make_async_copy(pa_h.at[pl.ds(0, CH)], rb, sss[b]).wait()

                    @pl.when(cid == 0)
                    def _():
                        pltpu.make_async_copy(ex_h.at[pl.ds(0, CH)], eb, sss[b]).wait()

                @pl.when(cid == 0)
                def _():
                    pltpu.async_copy(pa_h.at[pl.ds(gbase, CH)], rb, srs[b])

                @pl.when(cid == 1)
                def _():
                    pltpu.async_copy(pb_h.at[pl.ds(gbase, CH)], rb, srs[b])

                @pl.when(cid == 0)
                def _():
                    pltpu.async_copy(ex_h.at[pl.ds(gbase, CH)], eb, srs[b])

                pltpu.make_async_copy(pa_h.at[pl.ds(0, CH)], rb, srs[b]).wait()

                @pl.when(cid == 0)
                def _():
                    pltpu.make_async_copy(ex_h.at[pl.ds(0, CH)], eb, srs[b]).wait()

                idx = dss.at[ch]
                pltpu.async_copy(rb, acc_sh.at[idx], sss[b], add=True)

                @pl.when(cid == 0)
                def _():
                    pltpu.async_copy(eb, den_sh.at[idx], sss[b], add=True)

        for b in range(2):
            pltpu.make_async_copy(pa_h.at[pl.ds(0, CH)], rbufs[b], sss[b]).wait()

            @pl.when(cid == 0)
            def _():
                pltpu.make_async_copy(ex_h.at[pl.ds(0, CH)], ebufs[b], sss[b]).wait()

        plsc.subcore_barrier()

        @pl.when(sid < 15)
        def _():
            pltpu.sync_copy(acc_sh.at[pl.ds(sid * 640, 640)],
                            out_h.at[cid, pl.ds(sid * 640, 640)])

        @pl.when(sid == 15)
        def _():
            pltpu.sync_copy(acc_sh.at[pl.ds(9600, 400)],
                            out_h.at[cid, pl.ds(9600, 400)])

        @pl.when((sid == 0) & (cid == 0))
        def _():
            pltpu.sync_copy(den_sh, den_h)

    return accum_kernel


def _final_body(acc_ref, den_ref, res_ref, o_ref):
    den = den_ref[...] + 1e-16
    o_ref[:, :128] = jax.nn.relu(acc_ref[0] / den + res_ref[:, :128])
    o_ref[:, 128:] = jax.nn.relu(acc_ref[1] / den + res_ref[:, 128:])


def _run_final(acc2, den2, res):
    BN = 1000
    return pl.pallas_call(
        _final_body,
        grid=(N // BN,),
        in_specs=[
            pl.BlockSpec((NC, BN, 128), lambda i: (0, i, 0)),
            pl.BlockSpec((BN, 1), lambda i: (i, 0)),
            pl.BlockSpec((BN, D), lambda i: (i, 0)),
        ],
        out_specs=pl.BlockSpec((BN, D), lambda i: (i, 0)),
        out_shape=jax.ShapeDtypeStruct((N, D), jnp.float32),
    )(acc2, den2, res)


def kernel(h, edge_index, edge_attr, node_type, params):
    src = edge_index[0].astype(jnp.int32)
    dst = edge_index[1].astype(jnp.int32)
    nt = node_type.astype(jnp.int32)
    E = src.shape[0]
    EP = ((E + 4095) // 4096) * 4096
    pad = EP - E

    src_p = jnp.concatenate([src, jnp.zeros((pad,), jnp.int32)])
    dst_p = jnp.concatenate([dst, jnp.zeros((pad,), jnp.int32)])
    ea_p = jnp.concatenate([edge_attr, jnp.zeros((pad, ED), jnp.float32)])

    names = ("var", "red", "irr")
    Wl3 = jnp.stack([params[k]["Wl"] for k in names])
    bl3 = jnp.stack([params[k]["bl"] for k in names])
    Wr3 = jnp.stack([params[k]["Wr"] for k in names])
    br3 = jnp.stack([params[k]["br"] for k in names])
    We3 = jnp.stack([params[k]["We"] for k in names])
    att3 = jnp.stack([params[k]["att"] for k in names])
    Wres3 = jnp.stack([params[k]["Wres"] for k in names])
    bias3 = jnp.stack([params[k]["bias"] for k in names])

    tdst, gsrc = _make_idx_kernel(EP)(src_p, dst_p, nt)
    xl3, xr, res = _run_bnodes(
        h, nt.reshape(N, 1), Wl3, bl3, Wr3, br3, Wres3, bias3)
    xl = xl3.reshape(3 * N, D)

    xg, xrg = _make_gather_kernel(EP)(xl, xr, gsrc, dst_p)
    pa, pb, ex2 = _make_c2(EP, E)(xg, xrg, ea_p, tdst.reshape(EP, 1), att3, We3)

    dst3 = dst_p.reshape(NS, EP // NS // 64, 64)
    acc2, den = _make_accum_kernel(EP)(pa, pb, dst3, ex2.reshape(EP))
    return _run_final(acc2, den.reshape(N, 1), res)


# speedup vs baseline: 1.4673x; 1.2507x over previous
import dataclasses
import functools

import jax
import jax.numpy as jnp
from jax import lax
from jax.experimental import pallas as pl
from jax.experimental.pallas import tpu as pltpu
from jax.experimental.pallas import tpu_sc as plsc

N = 10000
D = 256
ED = 16
NC = 2
NS = 16
L = 16

_MESH = dict(mesh=plsc.VectorSubcoreMesh(core_axis_name="c", subcore_axis_name="s"))

_CP = pltpu.CompilerParams()
if "needs_layout_passes" in pltpu.CompilerParams.__dataclass_fields__:
    _CP = dataclasses.replace(_CP, needs_layout_passes=False)


def _make_idx_kernel(EP):
    per_w = EP // (NC * NS)
    n_ch = per_w // 128

    @functools.partial(
        pl.kernel,
        out_type=(
            jax.ShapeDtypeStruct((EP,), jnp.int32),
            jax.ShapeDtypeStruct((EP,), jnp.int32),
        ),
        scratch_types=[
            pltpu.VMEM((per_w,), jnp.int32),
            pltpu.VMEM((per_w,), jnp.int32),
            pltpu.VMEM((per_w,), jnp.int32),
            pltpu.VMEM((per_w,), jnp.int32),
            pltpu.VMEM((128,), jnp.int32),
            pltpu.VMEM((128,), jnp.int32),
            pltpu.SemaphoreType.DMA,
            pltpu.SemaphoreType.DMA,
        ],
        compiler_params=_CP,
        **_MESH,
    )
    def idx_kernel(src_h, dst_h, nt_h, tdst_h, gsrc_h,
                   dss, sss, tds, gss, td0, td1, sg0, sg1):
        wid = lax.axis_index("s") * NC + lax.axis_index("c")
        w0 = wid * per_w
        pltpu.sync_copy(dst_h.at[pl.ds(w0, per_w)], dss)
        pltpu.sync_copy(src_h.at[pl.ds(w0, per_w)], sss)
        tdb = (td0, td1)
        sgs = (sg0, sg1)

        @pl.loop(0, n_ch, step=2)
        def _(ch0):
            for b in range(2):
                off = (ch0 + b) * 128
                pltpu.async_copy(
                    nt_h.at[dss.at[pl.ds(off, 128)]], tdb[b], sgs[b])
            for b in range(2):
                off = (ch0 + b) * 128
                pltpu.make_async_copy(
                    nt_h.at[pl.ds(0, 128)], tdb[b], sgs[b]).wait()
                for v in range(8):
                    sl = pl.ds(off + v * L, L)
                    t = tdb[b][pl.ds(v * L, L)]
                    tds[sl] = t
                    gss[sl] = t * N + sss[sl]

        pltpu.sync_copy(tds, tdst_h.at[pl.ds(w0, per_w)])
        pltpu.sync_copy(gss, gsrc_h.at[pl.ds(w0, per_w)])

    return idx_kernel


def _bnodes_body(h_ref, nt_ref, wl_ref, bl_ref, wr_ref, br_ref, ws_ref, bs_ref,
                 xl_ref, xr_ref, res_ref):
    hb = h_ref[...]
    ntb = nt_ref[...]
    xr = []
    rs = []
    for b in range(3):
        xl_ref[b] = jnp.dot(hb, wl_ref[b], preferred_element_type=jnp.float32) + bl_ref[b][None, :]
        xr.append(jnp.dot(hb, wr_ref[b], preferred_element_type=jnp.float32) + br_ref[b][None, :])
        rs.append(jnp.dot(hb, ws_ref[b], preferred_element_type=jnp.float32) + bs_ref[b][None, :])
    xr_ref[...] = jnp.where(ntb == 0, xr[0], jnp.where(ntb == 1, xr[1], xr[2]))
    res_ref[...] = jnp.where(ntb == 0, rs[0], jnp.where(ntb == 1, rs[1], rs[2]))


def _run_bnodes(h, nt2, Wl3, bl3, Wr3, br3, Wres3, bias3):
    BN = 1000
    full = lambda shp: pl.BlockSpec(shp, lambda i: tuple(0 for _ in shp))
    return pl.pallas_call(
        _bnodes_body,
        grid=(N // BN,),
        in_specs=[
            pl.BlockSpec((BN, D), lambda i: (i, 0)),
            pl.BlockSpec((BN, 1), lambda i: (i, 0)),
            full((3, D, D)), full((3, D)), full((3, D, D)), full((3, D)),
            full((3, D, D)), full((3, D)),
        ],
        out_specs=[
            pl.BlockSpec((3, BN, D), lambda i: (0, i, 0)),
            pl.BlockSpec((BN, D), lambda i: (i, 0)),
            pl.BlockSpec((BN, D), lambda i: (i, 0)),
        ],
        out_shape=[
            jax.ShapeDtypeStruct((3, N, D), jnp.float32),
            jax.ShapeDtypeStruct((N, D), jnp.float32),
            jax.ShapeDtypeStruct((N, D), jnp.float32),
        ],
    )(h, nt2, Wl3, bl3, Wr3, br3, Wres3, bias3)


def _make_gather_kernel(EP):
    per_w = EP // (NC * NS)
    CH = 40
    n_ch = per_w // CH
    NB = 4

    @functools.partial(
        pl.kernel,
        out_type=(
            jax.ShapeDtypeStruct((EP, D), jnp.float32),
            jax.ShapeDtypeStruct((EP, D), jnp.float32),
        ),
        scratch_types=(
            [pltpu.VMEM((per_w,), jnp.int32)] * 2
            + [pltpu.VMEM((CH, D), jnp.float32)] * (2 * NB)
            + [pltpu.SemaphoreType.DMA] * (2 * NB)
        ),
        compiler_params=_CP,
        **_MESH,
    )
    def gather_kernel(xl_h, xr_h, gsrc_h, dst_h, gxl_h, gxr_h,
                      gss, dss, *bufsem):
        xlb = bufsem[0:NB]
        xrb = bufsem[NB:2 * NB]
        sg = bufsem[2 * NB:3 * NB]
        sw = bufsem[3 * NB:4 * NB]
        wid = lax.axis_index("s") * NC + lax.axis_index("c")
        w0 = wid * per_w
        pltpu.sync_copy(gsrc_h.at[pl.ds(w0, per_w)], gss)
        pltpu.sync_copy(dst_h.at[pl.ds(w0, per_w)], dss)

        def issue_g(k, s):
            off = k * CH
            pltpu.async_copy(xl_h.at[gss.at[pl.ds(off, CH)]], xlb[s], sg[s])
            pltpu.async_copy(xr_h.at[dss.at[pl.ds(off, CH)]], xrb[s], sg[s])

        issue_g(0, 0)
        issue_g(1, 1)

        @pl.loop(0, n_ch, step=NB)
        def _(k0):
            for j in range(NB):
                k = k0 + j
                s = j
                pltpu.make_async_copy(gxl_h.at[pl.ds(0, CH)], xlb[s], sg[s]).wait()
                pltpu.make_async_copy(gxl_h.at[pl.ds(0, CH)], xrb[s], sg[s]).wait()
                gout = w0 + k * CH
                pltpu.async_copy(xlb[s], gxl_h.at[pl.ds(gout, CH)], sw[s])
                pltpu.async_copy(xrb[s], gxr_h.at[pl.ds(gout, CH)], sw[s])
                s1 = (j + 3) % NB

                @pl.when(k >= 1)
                def _():
                    pltpu.make_async_copy(gxl_h.at[pl.ds(0, CH)], xlb[s1], sw[s1]).wait()
                    pltpu.make_async_copy(gxl_h.at[pl.ds(0, CH)], xrb[s1], sw[s1]).wait()

                @pl.when(k == 0)
                def _():
                    issue_g(2, (j + 2) % NB)
                    issue_g(3, (j + 3) % NB)

                @pl.when((k + 3 < n_ch) & (k >= 1))
                def _():
                    issue_g(k + 3, s1)

        s_last = (n_ch - 1) % NB
        pltpu.make_async_copy(gxl_h.at[pl.ds(0, CH)], xlb[s_last], sw[s_last]).wait()
        pltpu.make_async_copy(gxl_h.at[pl.ds(0, CH)], xrb[s_last], sw[s_last]).wait()

    return gather_kernel


def _make_c2(EP, E):
    BE = 2048

    def c2_body(xg_ref, xrg_ref, ea_ref, td_ref, att_ref, we_ref,
                pa_ref, pb_ref, ex_ref):
        i = pl.program_id(0)
        td = td_ref[...]
        att = jnp.where(
            td == 0, att_ref[0][None, :],
            jnp.where(td == 1, att_ref[1][None, :], att_ref[2][None, :]))
        eab = ea_ref[...]
        es = [jnp.dot(eab, we_ref[b], preferred_element_type=jnp.float32)
              for b in range(3)]
        e = jnp.where(td == 0, es[0], jnp.where(td == 1, es[1], es[2]))
        xg = xg_ref[...]
        m = xg + xrg_ref[...] + e
        m = jnp.maximum(m, m * 0.2)
        alpha = jnp.sum(m * att, axis=1, keepdims=True)
        ids = i * BE + lax.broadcasted_iota(jnp.int32, (BE, 1), 0)
        ex = jnp.where(ids < E, jnp.exp(alpha), 0.0)
        ex_ref[...] = ex
        pa_ref[...] = xg[:, :128] * ex
        pb_ref[...] = xg[:, 128:] * ex

    def run(xg, xrg, ea_p, td2, att3, We3):
        fullw = pl.BlockSpec((BE, D), lambda i: (i, 0))
        half = pl.BlockSpec((BE, 128), lambda i: (i, 0))
        return pl.pallas_call(
            c2_body,
            grid=(EP // BE,),
            in_specs=[fullw, fullw,
                      pl.BlockSpec((BE, ED), lambda i: (i, 0)),
                      pl.BlockSpec((BE, 1), lambda i: (i, 0)),
                      pl.BlockSpec((3, D), lambda i: (0, 0)),
                      pl.BlockSpec((3, ED, D), lambda i: (0, 0, 0))],
            out_specs=[half, half, pl.BlockSpec((BE, 1), lambda i: (i, 0))],
            out_shape=[
                jax.ShapeDtypeStruct((EP, 128), jnp.float32),
                jax.ShapeDtypeStruct((EP, 128), jnp.float32),
                jax.ShapeDtypeStruct((EP, 1), jnp.float32),
            ],
        )(xg, xrg, ea_p, td2, att3, We3)

    return run


def _make_accum_kernel(EP):
    per_s = EP // NS
    CH = 64
    n_ch = per_s // CH

    @functools.partial(
        pl.kernel,
        out_type=(
            jax.ShapeDtypeStruct((NC, N, 128), jnp.float32),
            jax.ShapeDtypeStruct((N,), jnp.float32),
        ),
        scratch_types=[
            pltpu.VMEM((n_ch, CH), jnp.int32),
            pltpu.VMEM((CH, 128), jnp.float32),
            pltpu.VMEM((CH, 128), jnp.float32),
            pltpu.VMEM((CH,), jnp.float32),
            pltpu.VMEM((CH,), jnp.float32),
            pltpu.VMEM((64, 128), jnp.float32),
            pltpu.VMEM((640,), jnp.float32),
            pltpu.SemaphoreType.DMA,
            pltpu.SemaphoreType.DMA,
            pltpu.SemaphoreType.DMA,
            pltpu.SemaphoreType.DMA,
            pltpu.VMEM_SHARED((N, 128), jnp.float32),
            pltpu.VMEM_SHARED((N,), jnp.float32),
        ],
        compiler_params=_CP,
        **_MESH,
    )
    def accum_kernel(pa_h, pb_h, dst3_h, ex_h, out_h, den_h,
                     dss, r0buf, r1buf, e0buf, e1buf, zb, zbd,
                     sr0, sr1, ss0, ss1,
                     acc_sh, den_sh):
        cid = lax.axis_index("c")
        sid = lax.axis_index("s")
        s0 = sid * per_s
        pltpu.sync_copy(dst3_h.at[sid], dss)

        @pl.when(sid == 0)
        def _():
            @pl.loop(0, 64)
            def _(r):
                for k in range(8):
                    zb[r, pl.ds(k * L, L)] = jnp.zeros((L,), jnp.float32)

            @pl.loop(0, 640 // L)
            def _(i):
                zbd[pl.ds(i * L, L)] = jnp.zeros((L,), jnp.float32)

            @pl.loop(0, 156)
            def _(i):
                pltpu.sync_copy(zb, acc_sh.at[pl.ds(i * 64, 64)])

            pltpu.sync_copy(zb.at[pl.ds(0, 16)], acc_sh.at[pl.ds(9984, 16)])

            @pl.loop(0, 15)
            def _(i):
                pltpu.sync_copy(zbd, den_sh.at[pl.ds(i * 640, 640)])

            pltpu.sync_copy(zbd.at[pl.ds(0, 400)], den_sh.at[pl.ds(9600, 400)])

        plsc.subcore_barrier()

        rbufs = (r0buf, r1buf)
        ebufs = (e0buf, e1buf)
        srs = (sr0, sr1)
        sss = (ss0, ss1)

        @pl.loop(0, n_ch, step=2)
        def _(ch0):
            for b in range(2):
                ch = ch0 + b
                gbase = s0 + ch * CH
                rb = rbufs[b]
                eb = ebufs[b]

                @pl.when(ch0 >= 2)
                def _():
                    pltpu.make_async_copy(pa_h.at[pl.ds(0, CH)], rb, sss[b]).wait()

                    @pl.when(cid == 0)
                    def _():
                        pltpu.make_async_copy(ex_h.at[pl.ds(0, CH)], eb, sss[b]).wait()

                @pl.when(cid == 0)
                def _():
                    pltpu.async_copy(pa_h.at[pl.ds(gbase, CH)], rb, srs[b])

                @pl.when(cid == 1)
                def _():
                    pltpu.async_copy(pb_h.at[pl.ds(gbase, CH)], rb, srs[b])

                @pl.when(cid == 0)
                def _():
                    pltpu.async_copy(ex_h.at[pl.ds(gbase, CH)], eb, srs[b])

                pltpu.make_async_copy(pa_h.at[pl.ds(0, CH)], rb, srs[b]).wait()

                @pl.when(cid == 0)
                def _():
                    pltpu.make_async_copy(ex_h.at[pl.ds(0, CH)], eb, srs[b]).wait()

                idx = dss.at[ch]
                pltpu.async_copy(rb, acc_sh.at[idx], sss[b], add=True)

                @pl.when(cid == 0)
                def _():
                    pltpu.async_copy(eb, den_sh.at[idx], sss[b], add=True)

        for b in range(2):
            pltpu.make_async_copy(pa_h.at[pl.ds(0, CH)], rbufs[b], sss[b]).wait()

            @pl.when(cid == 0)
            def _():
                pltpu.make_async_copy(ex_h.at[pl.ds(0, CH)], ebufs[b], sss[b]).wait()

        plsc.subcore_barrier()

        @pl.when(sid < 15)
        def _():
            pltpu.sync_copy(acc_sh.at[pl.ds(sid * 640, 640)],
                            out_h.at[cid, pl.ds(sid * 640, 640)])

        @pl.when(sid == 15)
        def _():
            pltpu.sync_copy(acc_sh.at[pl.ds(9600, 400)],
                            out_h.at[cid, pl.ds(9600, 400)])

        @pl.when((sid == 0) & (cid == 0))
        def _():
            pltpu.sync_copy(den_sh, den_h)

    return accum_kernel


def _final_body(acc_ref, den_ref, res_ref, o_ref):
    den = den_ref[...] + 1e-16
    o_ref[:, :128] = jax.nn.relu(acc_ref[0] / den + res_ref[:, :128])
    o_ref[:, 128:] = jax.nn.relu(acc_ref[1] / den + res_ref[:, 128:])


def _run_final(acc2, den2, res):
    BN = 1000
    return pl.pallas_call(
        _final_body,
        grid=(N // BN,),
        in_specs=[
            pl.BlockSpec((NC, BN, 128), lambda i: (0, i, 0)),
            pl.BlockSpec((BN, 1), lambda i: (i, 0)),
            pl.BlockSpec((BN, D), lambda i: (i, 0)),
        ],
        out_specs=pl.BlockSpec((BN, D), lambda i: (i, 0)),
        out_shape=jax.ShapeDtypeStruct((N, D), jnp.float32),
    )(acc2, den2, res)


def kernel(h, edge_index, edge_attr, node_type, params):
    src = edge_index[0].astype(jnp.int32)
    dst = edge_index[1].astype(jnp.int32)
    nt = node_type.astype(jnp.int32)
    E = src.shape[0]
    EP = ((E + 4095) // 4096) * 4096
    pad = EP - E

    spread = jnp.arange(pad, dtype=jnp.int32) % N
    src_p = jnp.concatenate([src, spread])
    dst_p = jnp.concatenate([dst, spread])
    ea_p = jnp.concatenate([edge_attr, jnp.zeros((pad, ED), jnp.float32)])

    names = ("var", "red", "irr")
    Wl3 = jnp.stack([params[k]["Wl"] for k in names])
    bl3 = jnp.stack([params[k]["bl"] for k in names])
    Wr3 = jnp.stack([params[k]["Wr"] for k in names])
    br3 = jnp.stack([params[k]["br"] for k in names])
    We3 = jnp.stack([params[k]["We"] for k in names])
    att3 = jnp.stack([params[k]["att"] for k in names])
    Wres3 = jnp.stack([params[k]["Wres"] for k in names])
    bias3 = jnp.stack([params[k]["bias"] for k in names])

    tdst, gsrc = _make_idx_kernel(EP)(src_p, dst_p, nt)
    xl3, xr, res = _run_bnodes(
        h, nt.reshape(N, 1), Wl3, bl3, Wr3, br3, Wres3, bias3)
    xl = xl3.reshape(3 * N, D)

    xg, xrg = _make_gather_kernel(EP)(xl, xr, gsrc, dst_p)
    pa, pb, ex2 = _make_c2(EP, E)(xg, xrg, ea_p, tdst.reshape(EP, 1), att3, We3)

    dst3 = dst_p.reshape(NS, EP // NS // 64, 64)
    acc2, den = _make_accum_kernel(EP)(pa, pb, dst3, ex2.reshape(EP))
    return _run_final(acc2, den.reshape(N, 1), res)


# speedup vs baseline: 1.5613x; 1.0641x over previous
import dataclasses
import functools

import jax
import jax.numpy as jnp
from jax import lax
from jax.experimental import pallas as pl
from jax.experimental.pallas import tpu as pltpu
from jax.experimental.pallas import tpu_sc as plsc

N = 10000
D = 256
ED = 16
NC = 2
NS = 16
L = 16

_MESH = dict(mesh=plsc.VectorSubcoreMesh(core_axis_name="c", subcore_axis_name="s"))

_CP = pltpu.CompilerParams()
if "needs_layout_passes" in pltpu.CompilerParams.__dataclass_fields__:
    _CP = dataclasses.replace(_CP, needs_layout_passes=False)


def _make_idx_kernel(EP):
    per_w = EP // (NC * NS)
    n_ch = per_w // 128

    @functools.partial(
        pl.kernel,
        out_type=(
            jax.ShapeDtypeStruct((EP,), jnp.int32),
            jax.ShapeDtypeStruct((EP,), jnp.int32),
        ),
        scratch_types=[
            pltpu.VMEM((per_w,), jnp.int32),
            pltpu.VMEM((per_w,), jnp.int32),
            pltpu.VMEM((per_w,), jnp.int32),
            pltpu.VMEM((per_w,), jnp.int32),
            pltpu.VMEM((128,), jnp.int32),
            pltpu.VMEM((128,), jnp.int32),
            pltpu.SemaphoreType.DMA,
            pltpu.SemaphoreType.DMA,
        ],
        compiler_params=_CP,
        **_MESH,
    )
    def idx_kernel(src_h, dst_h, nt_h, tdst_h, gsrc_h,
                   dss, sss, tds, gss, td0, td1, sg0, sg1):
        wid = lax.axis_index("s") * NC + lax.axis_index("c")
        w0 = wid * per_w
        pltpu.sync_copy(dst_h.at[pl.ds(w0, per_w)], dss)
        pltpu.sync_copy(src_h.at[pl.ds(w0, per_w)], sss)
        tdb = (td0, td1)
        sgs = (sg0, sg1)

        @pl.loop(0, n_ch, step=2)
        def _(ch0):
            for b in range(2):
                off = (ch0 + b) * 128
                pltpu.async_copy(
                    nt_h.at[dss.at[pl.ds(off, 128)]], tdb[b], sgs[b])
            for b in range(2):
                off = (ch0 + b) * 128
                pltpu.make_async_copy(
                    nt_h.at[pl.ds(0, 128)], tdb[b], sgs[b]).wait()
                for v in range(8):
                    sl = pl.ds(off + v * L, L)
                    t = tdb[b][pl.ds(v * L, L)]
                    tds[sl] = t
                    gss[sl] = t * N + sss[sl]

        pltpu.sync_copy(tds, tdst_h.at[pl.ds(w0, per_w)])
        pltpu.sync_copy(gss, gsrc_h.at[pl.ds(w0, per_w)])

    return idx_kernel


def _bnodes_body(h_ref, nt_ref, wl_ref, bl_ref, wr_ref, br_ref, ws_ref, bs_ref,
                 xl_ref, xr_ref, res_ref):
    hb = h_ref[...]
    ntb = nt_ref[...]
    xr = []
    rs = []
    for b in range(3):
        xl_ref[b] = jnp.dot(hb, wl_ref[b], preferred_element_type=jnp.float32) + bl_ref[b][None, :]
        xr.append(jnp.dot(hb, wr_ref[b], preferred_element_type=jnp.float32) + br_ref[b][None, :])
        rs.append(jnp.dot(hb, ws_ref[b], preferred_element_type=jnp.float32) + bs_ref[b][None, :])
    xr_ref[...] = jnp.where(ntb == 0, xr[0], jnp.where(ntb == 1, xr[1], xr[2]))
    res_ref[...] = jnp.where(ntb == 0, rs[0], jnp.where(ntb == 1, rs[1], rs[2]))


def _run_bnodes(h, nt2, Wl3, bl3, Wr3, br3, Wres3, bias3):
    BN = 1000
    full = lambda shp: pl.BlockSpec(shp, lambda i: tuple(0 for _ in shp))
    return pl.pallas_call(
        _bnodes_body,
        grid=(N // BN,),
        in_specs=[
            pl.BlockSpec((BN, D), lambda i: (i, 0)),
            pl.BlockSpec((BN, 1), lambda i: (i, 0)),
            full((3, D, D)), full((3, D)), full((3, D, D)), full((3, D)),
            full((3, D, D)), full((3, D)),
        ],
        out_specs=[
            pl.BlockSpec((3, BN, D), lambda i: (0, i, 0)),
            pl.BlockSpec((BN, D), lambda i: (i, 0)),
            pl.BlockSpec((BN, D), lambda i: (i, 0)),
        ],
        out_shape=[
            jax.ShapeDtypeStruct((3, N, D), jnp.float32),
            jax.ShapeDtypeStruct((N, D), jnp.float32),
            jax.ShapeDtypeStruct((N, D), jnp.float32),
        ],
    )(h, nt2, Wl3, bl3, Wr3, br3, Wres3, bias3)


def _make_gather_kernel(EP):
    per_w = EP // (NC * NS)
    CH = 40
    n_ch = per_w // CH
    NB = 4

    @functools.partial(
        pl.kernel,
        out_type=(
            jax.ShapeDtypeStruct((EP, D), jnp.float32),
            jax.ShapeDtypeStruct((EP, D), jnp.float32),
        ),
        scratch_types=(
            [pltpu.VMEM((per_w,), jnp.int32)] * 2
            + [pltpu.VMEM((CH, D), jnp.float32)] * (2 * NB)
            + [pltpu.SemaphoreType.DMA] * (2 * NB)
        ),
        compiler_params=_CP,
        **_MESH,
    )
    def gather_kernel(xl_h, xr_h, gsrc_h, dst_h, gxl_h, gxr_h,
                      gss, dss, *bufsem):
        xlb = bufsem[0:NB]
        xrb = bufsem[NB:2 * NB]
        sg = bufsem[2 * NB:3 * NB]
        sw = bufsem[3 * NB:4 * NB]
        wid = lax.axis_index("s") * NC + lax.axis_index("c")
        w0 = wid * per_w
        pltpu.sync_copy(gsrc_h.at[pl.ds(w0, per_w)], gss)
        pltpu.sync_copy(dst_h.at[pl.ds(w0, per_w)], dss)

        def issue_g(k, s):
            off = k * CH
            pltpu.async_copy(xl_h.at[gss.at[pl.ds(off, CH)]], xlb[s], sg[s])
            pltpu.async_copy(xr_h.at[dss.at[pl.ds(off, CH)]], xrb[s], sg[s])

        issue_g(0, 0)
        issue_g(1, 1)

        @pl.loop(0, n_ch, step=NB)
        def _(k0):
            for j in range(NB):
                k = k0 + j
                s = j
                pltpu.make_async_copy(gxl_h.at[pl.ds(0, CH)], xlb[s], sg[s]).wait()
                pltpu.make_async_copy(gxl_h.at[pl.ds(0, CH)], xrb[s], sg[s]).wait()
                gout = w0 + k * CH
                pltpu.async_copy(xlb[s], gxl_h.at[pl.ds(gout, CH)], sw[s])
                pltpu.async_copy(xrb[s], gxr_h.at[pl.ds(gout, CH)], sw[s])
                s1 = (j + 3) % NB

                @pl.when(k >= 1)
                def _():
                    pltpu.make_async_copy(gxl_h.at[pl.ds(0, CH)], xlb[s1], sw[s1]).wait()
                    pltpu.make_async_copy(gxl_h.at[pl.ds(0, CH)], xrb[s1], sw[s1]).wait()

                @pl.when(k == 0)
                def _():
                    issue_g(2, (j + 2) % NB)
                    issue_g(3, (j + 3) % NB)

                @pl.when((k + 3 < n_ch) & (k >= 1))
                def _():
                    issue_g(k + 3, s1)

        s_last = (n_ch - 1) % NB
        pltpu.make_async_copy(gxl_h.at[pl.ds(0, CH)], xlb[s_last], sw[s_last]).wait()
        pltpu.make_async_copy(gxl_h.at[pl.ds(0, CH)], xrb[s_last], sw[s_last]).wait()

    return gather_kernel


def _make_c2(EP, E):
    BE = 2048

    def c2_body(xg_ref, xrg_ref, ea_ref, td_ref, att_ref, we_ref,
                pa_ref, pb_ref, ex_ref):
        i = pl.program_id(0)
        td = td_ref[...]
        att = jnp.where(
            td == 0, att_ref[0][None, :],
            jnp.where(td == 1, att_ref[1][None, :], att_ref[2][None, :]))
        eab = ea_ref[...]
        es = [jnp.dot(eab, we_ref[b], preferred_element_type=jnp.float32)
              for b in range(3)]
        e = jnp.where(td == 0, es[0], jnp.where(td == 1, es[1], es[2]))
        xg = xg_ref[...]
        m = xg + xrg_ref[...] + e
        m = jnp.maximum(m, m * 0.2)
        alpha = jnp.sum(m * att, axis=1, keepdims=True)
        ids = i * BE + lax.broadcasted_iota(jnp.int32, (BE, 1), 0)
        ex = jnp.where(ids < E, jnp.exp(alpha), 0.0)
        ex_ref[...] = ex
        pa_ref[...] = xg[:, :128] * ex
        pb_ref[...] = xg[:, 128:] * ex

    def run(xg, xrg, ea_p, td2, att3, We3):
        fullw = pl.BlockSpec((BE, D), lambda i: (i, 0))
        half = pl.BlockSpec((BE, 128), lambda i: (i, 0))
        return pl.pallas_call(
            c2_body,
            grid=(EP // BE,),
            in_specs=[fullw, fullw,
                      pl.BlockSpec((BE, ED), lambda i: (i, 0)),
                      pl.BlockSpec((BE, 1), lambda i: (i, 0)),
                      pl.BlockSpec((3, D), lambda i: (0, 0)),
                      pl.BlockSpec((3, ED, D), lambda i: (0, 0, 0))],
            out_specs=[half, half, pl.BlockSpec((BE, 1), lambda i: (i, 0))],
            out_shape=[
                jax.ShapeDtypeStruct((EP, 128), jnp.float32),
                jax.ShapeDtypeStruct((EP, 128), jnp.float32),
                jax.ShapeDtypeStruct((EP, 1), jnp.float32),
            ],
        )(xg, xrg, ea_p, td2, att3, We3)

    return run


def _make_accum_kernel(EP):
    per_s = EP // NS
    CH = 32
    n_ch = per_s // CH
    NB = 4

    @functools.partial(
        pl.kernel,
        out_type=(
            jax.ShapeDtypeStruct((NC, N, 128), jnp.float32),
            jax.ShapeDtypeStruct((N,), jnp.float32),
        ),
        scratch_types=(
            [pltpu.VMEM((CH, 128), jnp.float32)] * NB
            + [pltpu.VMEM((CH,), jnp.float32)] * NB
            + [pltpu.VMEM((CH,), jnp.int32)] * NB
            + [pltpu.VMEM((32, 128), jnp.float32)]
            + [pltpu.VMEM((640,), jnp.float32)]
            + [pltpu.SemaphoreType.DMA] * (2 * NB)
            + [pltpu.VMEM_SHARED((N, 128), jnp.float32)]
            + [pltpu.VMEM_SHARED((N,), jnp.float32)]
        ),
        compiler_params=_CP,
        **_MESH,
    )
    def accum_kernel(pa_h, pb_h, dst_h, ex_h, out_h, den_h, *rest):
        rbufs = rest[0:NB]
        ebufs = rest[NB:2 * NB]
        ibufs = rest[2 * NB:3 * NB]
        zb = rest[3 * NB]
        zbd = rest[3 * NB + 1]
        srs = rest[3 * NB + 2:3 * NB + 2 + NB]
        sss = rest[3 * NB + 2 + NB:3 * NB + 2 + 2 * NB]
        acc_sh = rest[3 * NB + 2 + 2 * NB]
        den_sh = rest[3 * NB + 3 + 2 * NB]
        cid = lax.axis_index("c")
        sid = lax.axis_index("s")
        s0 = sid * per_s

        @pl.when(sid == 0)
        def _():
            @pl.loop(0, 32)
            def _(r):
                for k in range(8):
                    zb[r, pl.ds(k * L, L)] = jnp.zeros((L,), jnp.float32)

            @pl.loop(0, 640 // L)
            def _(i):
                zbd[pl.ds(i * L, L)] = jnp.zeros((L,), jnp.float32)

            @pl.loop(0, 312)
            def _(i):
                pltpu.sync_copy(zb, acc_sh.at[pl.ds(i * 32, 32)])

            pltpu.sync_copy(zb.at[pl.ds(0, 16)], acc_sh.at[pl.ds(9984, 16)])

            @pl.loop(0, 15)
            def _(i):
                pltpu.sync_copy(zbd, den_sh.at[pl.ds(i * 640, 640)])

            pltpu.sync_copy(zbd.at[pl.ds(0, 400)], den_sh.at[pl.ds(9600, 400)])

        plsc.subcore_barrier()

        def issue_r(k, s):
            gbase = s0 + k * CH
            pltpu.async_copy(dst_h.at[pl.ds(gbase, CH)], ibufs[s], srs[s])

            @pl.when(cid == 0)
            def _():
                pltpu.async_copy(pa_h.at[pl.ds(gbase, CH)], rbufs[s], srs[s])
                pltpu.async_copy(ex_h.at[pl.ds(gbase, CH)], ebufs[s], srs[s])

            @pl.when(cid == 1)
            def _():
                pltpu.async_copy(pb_h.at[pl.ds(gbase, CH)], rbufs[s], srs[s])

        issue_r(0, 0)
        issue_r(1, 1)

        @pl.loop(0, n_ch, step=NB)
        def _(k0):
            for j in range(NB):
                k = k0 + j
                s = j
                pltpu.make_async_copy(pa_h.at[pl.ds(0, CH)], rbufs[s], srs[s]).wait()
                pltpu.make_async_copy(dst_h.at[pl.ds(0, CH)], ibufs[s], srs[s]).wait()

                @pl.when(cid == 0)
                def _():
                    pltpu.make_async_copy(ex_h.at[pl.ds(0, CH)], ebufs[s], srs[s]).wait()

                idx = ibufs[s]
                pltpu.async_copy(rbufs[s], acc_sh.at[idx], sss[s], add=True)

                @pl.when(cid == 0)
                def _():
                    pltpu.async_copy(ebufs[s], den_sh.at[idx], sss[s], add=True)

                s1 = (j + 3) % NB

                @pl.when(k >= 1)
                def _():
                    pltpu.make_async_copy(pa_h.at[pl.ds(0, CH)], rbufs[s1], sss[s1]).wait()

                    @pl.when(cid == 0)
                    def _():
                        pltpu.make_async_copy(ex_h.at[pl.ds(0, CH)], ebufs[s1], sss[s1]).wait()

                @pl.when(k == 0)
                def _():
                    issue_r(2, (j + 2) % NB)
                    issue_r(3, (j + 3) % NB)

                @pl.when((k + 3 < n_ch) & (k >= 1))
                def _():
                    issue_r(k + 3, s1)

        s_last = (n_ch - 1) % NB
        pltpu.make_async_copy(pa_h.at[pl.ds(0, CH)], rbufs[s_last], sss[s_last]).wait()

        @pl.when(cid == 0)
        def _():
            pltpu.make_async_copy(ex_h.at[pl.ds(0, CH)], ebufs[s_last], sss[s_last]).wait()

        plsc.subcore_barrier()

        @pl.when(sid < 15)
        def _():
            pltpu.sync_copy(acc_sh.at[pl.ds(sid * 640, 640)],
                            out_h.at[cid, pl.ds(sid * 640, 640)])

        @pl.when(sid == 15)
        def _():
            pltpu.sync_copy(acc_sh.at[pl.ds(9600, 400)],
                            out_h.at[cid, pl.ds(9600, 400)])

        @pl.when((sid == 0) & (cid == 0))
        def _():
            pltpu.sync_copy(den_sh, den_h)

    return accum_kernel


def _final_body(acc_ref, den_ref, res_ref, o_ref):
    den = den_ref[...] + 1e-16
    o_ref[:, :128] = jax.nn.relu(acc_ref[0] / den + res_ref[:, :128])
    o_ref[:, 128:] = jax.nn.relu(acc_ref[1] / den + res_ref[:, 128:])


def _run_final(acc2, den2, res):
    BN = 1000
    return pl.pallas_call(
        _final_body,
        grid=(N // BN,),
        in_specs=[
            pl.BlockSpec((NC, BN, 128), lambda i: (0, i, 0)),
            pl.BlockSpec((BN, 1), lambda i: (i, 0)),
            pl.BlockSpec((BN, D), lambda i: (i, 0)),
        ],
        out_specs=pl.BlockSpec((BN, D), lambda i: (i, 0)),
        out_shape=jax.ShapeDtypeStruct((N, D), jnp.float32),
    )(acc2, den2, res)


def kernel(h, edge_index, edge_attr, node_type, params):
    src = edge_index[0].astype(jnp.int32)
    dst = edge_index[1].astype(jnp.int32)
    nt = node_type.astype(jnp.int32)
    E = src.shape[0]
    EP = ((E + 4095) // 4096) * 4096
    pad = EP - E

    spread = jnp.arange(pad, dtype=jnp.int32) % N
    src_p = jnp.concatenate([src, spread])
    dst_p = jnp.concatenate([dst, spread])
    ea_p = jnp.concatenate([edge_attr, jnp.zeros((pad, ED), jnp.float32)])

    names = ("var", "red", "irr")
    Wl3 = jnp.stack([params[k]["Wl"] for k in names])
    bl3 = jnp.stack([params[k]["bl"] for k in names])
    Wr3 = jnp.stack([params[k]["Wr"] for k in names])
    br3 = jnp.stack([params[k]["br"] for k in names])
    We3 = jnp.stack([params[k]["We"] for k in names])
    att3 = jnp.stack([params[k]["att"] for k in names])
    Wres3 = jnp.stack([params[k]["Wres"] for k in names])
    bias3 = jnp.stack([params[k]["bias"] for k in names])

    tdst, gsrc = _make_idx_kernel(EP)(src_p, dst_p, nt)
    xl3, xr, res = _run_bnodes(
        h, nt.reshape(N, 1), Wl3, bl3, Wr3, br3, Wres3, bias3)
    xl = xl3.reshape(3 * N, D)

    xg, xrg = _make_gather_kernel(EP)(xl, xr, gsrc, dst_p)
    pa, pb, ex2 = _make_c2(EP, E)(xg, xrg, ea_p, tdst.reshape(EP, 1), att3, We3)

    acc2, den = _make_accum_kernel(EP)(pa, pb, dst_p, ex2.reshape(EP))
    return _run_final(acc2, den.reshape(N, 1), res)
